# Initial kernel scaffold; baseline (speedup 1.0000x reference)
import jax
import jax.numpy as jnp
from jax.experimental import pallas as pl


def kernel(keys, values, access_costs):
    raise NotImplementedError("write your pallas kernel here")


# speedup vs baseline: 6.6543x; 6.6543x over previous
import jax
import jax.numpy as jnp
from jax import lax
from jax.experimental import pallas as pl
from jax.experimental.pallas import tpu as pltpu

_CACHE_SIZE = 24576
_CACHE_LEN = 32768
_K = _CACHE_LEN - _CACHE_SIZE

_ROWS = _CACHE_SIZE // 128
_OUT_ROWS = _CACHE_LEN // 128


def _select_body(costs_ref, out_ref):
    costs = costs_ref[...]
    mb = 1.0 / (costs + jnp.float32(1e-8))
    bits = lax.bitcast_convert_type(mb, jnp.int32)
    k = jnp.int32(_K)

    def bs1(_, carry):
        lo, hi = carry
        mid = lo + (hi - lo) // 2
        ge = jnp.sum((bits <= mid).astype(jnp.int32)) >= k
        return (jnp.where(ge, lo, mid), jnp.where(ge, mid, hi))

    lo, hi = lax.fori_loop(0, 31, bs1, (jnp.int32(-1), jnp.int32(0x7F7FFFFF)))
    w = hi
    lt = bits < w
    eq = bits == w
    r = k - jnp.sum(lt.astype(jnp.int32))

    row = lax.broadcasted_iota(jnp.int32, (_ROWS, 128), 0)
    col = lax.broadcasted_iota(jnp.int32, (_ROWS, 128), 1)
    idx = row * 128 + col

    def bs2(_, carry):
        lo2, hi2 = carry
        mid = lo2 + (hi2 - lo2) // 2
        ge = jnp.sum((eq & (idx < mid)).astype(jnp.int32)) >= r
        return (jnp.where(ge, lo2, mid), jnp.where(ge, mid, hi2))

    lo2, hi2 = lax.fori_loop(0, 15, bs2, (jnp.int32(0), jnp.int32(_CACHE_SIZE)))
    mask = lt | (eq & (idx < hi2))

    out_ref[...] = jnp.zeros((_OUT_ROWS, 128), jnp.int32)
    out_ref[0:_ROWS, :] = mask.astype(jnp.int32)


def kernel(keys, values, access_costs):
    del keys, values
    costs = access_costs.reshape(_ROWS, 128)
    out = pl.pallas_call(
        _select_body,
        out_shape=jax.ShapeDtypeStruct((_OUT_ROWS, 128), jnp.int32),
    )(costs)
    return out.reshape(_CACHE_LEN).astype(jnp.bool_)
